# baseline (speedup 1.0000x reference)
import numpy as np

import jax
import jax.numpy as jnp
from jax.experimental import pallas as pl
from jax.experimental.pallas import tpu as pltpu

N_Q = 8
BINS = 1024
DIM = 256
BATCH_PER_TILE = 2


def _rvq_body(x_ref, cb_ref, q_out_ref, codes_ref, c2_ref, cbh_ref, cbm_ref,
              cbl_ref):
    @pl.when(pl.program_id(0) == 0)
    def _():
        for i in range(N_Q):
            cbi = cb_ref[i]
            c2_ref[i] = 0.5 * jnp.sum(cbi * cbi, axis=1, keepdims=True)
            hi = cbi.astype(jnp.bfloat16)
            r1 = cbi - hi.astype(jnp.float32)
            mid = r1.astype(jnp.bfloat16)
            cbh_ref[i] = hi
            cbm_ref[i] = mid
            cbl_ref[i] = (r1 - mid.astype(jnp.float32)).astype(jnp.bfloat16)

    x0 = x_ref[...]
    t = x0.shape[2]
    iota = jax.lax.broadcasted_iota(jnp.int32, (BINS, t), 0)
    dn = (((1,), (0,)), ((), ()))
    dn_lt = (((0,), (0,)), ((), ()))
    rs = [x0[h] for h in range(BATCH_PER_TILE)]
    for i in range(N_Q):
        cb = cb_ref[i]
        for h in range(BATCH_PER_TILE):
            cross = jax.lax.dot_general(
                cb, rs[h], dn, preferred_element_type=jnp.float32)
            score = cross - c2_ref[i]
            idx = jnp.argmax(score, axis=0).astype(jnp.int32)
            codes_ref[i, pl.ds(h * t, t)] = idx
            onehot = (iota == idx[None, :]).astype(jnp.bfloat16)
            q = ((jax.lax.dot_general(cbh_ref[i], onehot, dn_lt,
                                      preferred_element_type=jnp.float32)
                  + jax.lax.dot_general(cbm_ref[i], onehot, dn_lt,
                                        preferred_element_type=jnp.float32))
                 + jax.lax.dot_general(cbl_ref[i], onehot, dn_lt,
                                       preferred_element_type=jnp.float32))
            rs[h] = rs[h] - q
    q_out_ref[...] = x0 - jnp.stack(rs, axis=0)


def kernel(x, codebooks, frame_rate):
    b, d, t = x.shape
    n_q, bins, dim = codebooks.shape

    grid = (b // BATCH_PER_TILE,)
    quantized, codes2d = pl.pallas_call(
        _rvq_body,
        grid=grid,
        in_specs=[
            pl.BlockSpec((BATCH_PER_TILE, d, t), lambda i: (i, 0, 0)),
            pl.BlockSpec((n_q, bins, dim), lambda i: (0, 0, 0)),
        ],
        out_specs=[
            pl.BlockSpec((BATCH_PER_TILE, d, t), lambda i: (i, 0, 0)),
            pl.BlockSpec((n_q, BATCH_PER_TILE * t), lambda i: (0, i)),
        ],
        out_shape=[
            jax.ShapeDtypeStruct((b, d, t), jnp.float32),
            jax.ShapeDtypeStruct((n_q, b * t), jnp.int32),
        ],
        scratch_shapes=[
            pltpu.VMEM((n_q, bins, 1), jnp.float32),
            pltpu.VMEM((n_q, bins, dim), jnp.bfloat16),
            pltpu.VMEM((n_q, bins, dim), jnp.bfloat16),
            pltpu.VMEM((n_q, bins, dim), jnp.bfloat16),
        ],
        compiler_params=pltpu.CompilerParams(
            dimension_semantics=("arbitrary",)),
    )(x, codebooks)

    codes = codes2d.reshape(n_q, b, t)
    bw = jnp.asarray(n_q * np.log2(bins) * frame_rate, dtype=x.dtype)
    return quantized, codes, bw


# speedup vs baseline: 1.0095x; 1.0095x over previous
import numpy as np

import jax
import jax.numpy as jnp
from jax.experimental import pallas as pl
from jax.experimental.pallas import tpu as pltpu

N_Q = 8
BINS = 1024
DIM = 256
BATCH_PER_TILE = 2


def _rvq_body(x_ref, cb_ref, q_out_ref, codes_ref, c2_ref, cbh_ref, cbm_ref,
              cbl_ref):
    @pl.when(pl.program_id(0) == 0)
    def _():
        for i in range(N_Q):
            cbi = cb_ref[i]
            c2_ref[i] = 0.5 * jnp.sum(cbi * cbi, axis=1, keepdims=True)
            cbit = cbi.T
            hi = cbit.astype(jnp.bfloat16)
            r1 = cbit - hi.astype(jnp.float32)
            mid = r1.astype(jnp.bfloat16)
            cbh_ref[i] = hi
            cbm_ref[i] = mid
            cbl_ref[i] = (r1 - mid.astype(jnp.float32)).astype(jnp.bfloat16)

    x0 = x_ref[...]
    t = x0.shape[2]
    iota = jax.lax.broadcasted_iota(jnp.int32, (BINS, t), 0)
    dn = (((1,), (0,)), ((), ()))
    dn_g = (((1,), (0,)), ((), ()))
    rs = [x0[h] for h in range(BATCH_PER_TILE)]
    for i in range(N_Q):
        cb = cb_ref[i]
        for h in range(BATCH_PER_TILE):
            cross = jax.lax.dot_general(
                cb, rs[h], dn, preferred_element_type=jnp.float32)
            score = cross - c2_ref[i]
            idx = jnp.argmax(score, axis=0).astype(jnp.int32)
            codes_ref[i, pl.ds(h * t, t)] = idx
            onehot = (iota == idx[None, :]).astype(jnp.bfloat16)
            q = ((jax.lax.dot_general(cbh_ref[i], onehot, dn_g,
                                      preferred_element_type=jnp.float32)
                  + jax.lax.dot_general(cbm_ref[i], onehot, dn_g,
                                        preferred_element_type=jnp.float32))
                 + jax.lax.dot_general(cbl_ref[i], onehot, dn_g,
                                       preferred_element_type=jnp.float32))
            rs[h] = rs[h] - q
    q_out_ref[...] = x0 - jnp.stack(rs, axis=0)


def kernel(x, codebooks, frame_rate):
    b, d, t = x.shape
    n_q, bins, dim = codebooks.shape

    grid = (b // BATCH_PER_TILE,)
    quantized, codes2d = pl.pallas_call(
        _rvq_body,
        grid=grid,
        in_specs=[
            pl.BlockSpec((BATCH_PER_TILE, d, t), lambda i: (i, 0, 0)),
            pl.BlockSpec((n_q, bins, dim), lambda i: (0, 0, 0)),
        ],
        out_specs=[
            pl.BlockSpec((BATCH_PER_TILE, d, t), lambda i: (i, 0, 0)),
            pl.BlockSpec((n_q, BATCH_PER_TILE * t), lambda i: (0, i)),
        ],
        out_shape=[
            jax.ShapeDtypeStruct((b, d, t), jnp.float32),
            jax.ShapeDtypeStruct((n_q, b * t), jnp.int32),
        ],
        scratch_shapes=[
            pltpu.VMEM((n_q, bins, 1), jnp.float32),
            pltpu.VMEM((n_q, dim, bins), jnp.bfloat16),
            pltpu.VMEM((n_q, dim, bins), jnp.bfloat16),
            pltpu.VMEM((n_q, dim, bins), jnp.bfloat16),
        ],
        compiler_params=pltpu.CompilerParams(
            dimension_semantics=("arbitrary",)),
    )(x, codebooks)

    codes = codes2d.reshape(n_q, b, t)
    bw = jnp.asarray(n_q * np.log2(bins) * frame_rate, dtype=x.dtype)
    return quantized, codes, bw
